# baseline (speedup 1.0000x reference)
import jax
import jax.numpy as jnp
from jax.experimental import pallas as pl

NC = 1000
ROWS = 4096 * 26
BLK = 512


def _onehot_body(x_ref, out_ref):
    idx = x_ref[0, 0, :]
    cls = jax.lax.broadcasted_iota(jnp.int32, (BLK, NC), 1)
    out_ref[...] = (idx[:, None] == cls).astype(jnp.float32)


def kernel(x):
    xf = x.reshape(ROWS // BLK, 1, BLK)
    out = pl.pallas_call(
        _onehot_body,
        grid=(ROWS // BLK,),
        in_specs=[pl.BlockSpec((1, 1, BLK), lambda i: (i, 0, 0))],
        out_specs=pl.BlockSpec((BLK, NC), lambda i: (i, 0)),
        out_shape=jax.ShapeDtypeStruct((ROWS, NC), jnp.float32),
    )(xf)
    return out.reshape(4096, 26, NC)
